# baseline (speedup 1.0000x reference)
import functools

import jax
import jax.numpy as jnp
import numpy as np
from jax.experimental import pallas as pl
from jax.experimental.pallas import tpu as pltpu

SR = 44100
N_FFT = 2048
HOP = 512
N_MELS = 128
D_MODEL = 512
VOCAB = 4096
N_CB = 4

ROWS_PAD = 1032
VCHUNK = 4096
NVC = VOCAB // VCHUNK


def _mel_filterbank():
    n_freqs = N_FFT // 2 + 1
    all_freqs = np.linspace(0.0, SR / 2.0, n_freqs)

    def hz2mel(f):
        return 2595.0 * np.log10(1.0 + f / 700.0)

    def mel2hz(m):
        return 700.0 * (10.0 ** (m / 2595.0) - 1.0)

    m_pts = np.linspace(hz2mel(0.0), hz2mel(SR / 2.0), N_MELS + 2)
    f_pts = mel2hz(m_pts)
    f_diff = f_pts[1:] - f_pts[:-1]
    slopes = f_pts[None, :] - all_freqs[:, None]
    down = -slopes[:, :-2] / f_diff[:-1]
    up = slopes[:, 2:] / f_diff[1:]
    fb = np.maximum(0.0, np.minimum(down, up))
    return fb.astype(np.float32)


NFRAMES = 257
NHOPS = 260
KFREQ = 1152

HI = jax.lax.Precision.HIGHEST


@functools.lru_cache(maxsize=1)
def _dft_consts():
    n = np.arange(N_FFT, dtype=np.float64)
    k = np.arange(KFREQ, dtype=np.float64)
    win = 0.5 - 0.5 * np.cos(2.0 * np.pi * n / N_FFT)
    ang = (2.0 * np.pi / N_FFT) * np.outer(n, k)
    cw = (win[:, None] * np.cos(ang)).astype(np.float32)
    sw = (win[:, None] * np.sin(ang)).astype(np.float32)
    cw[:, N_FFT // 2 + 1:] = 0.0
    sw[:, N_FFT // 2 + 1:] = 0.0
    fb = np.zeros((KFREQ, N_MELS), np.float32)
    fb[:N_FFT // 2 + 1] = _mel_filterbank()
    return cw, sw, fb


def _dot_split(a, b):
    a_hi = a.astype(jnp.bfloat16)
    a_lo = (a - a_hi.astype(jnp.float32)).astype(jnp.bfloat16)
    b_hi = b.astype(jnp.bfloat16)
    b_lo = (b - b_hi.astype(jnp.float32)).astype(jnp.bfloat16)

    def d(x, y):
        return jnp.dot(x, y, preferred_element_type=jnp.float32)

    return ((d(a_lo, b_lo) + d(a_lo, b_hi)) + d(a_hi, b_lo)) + d(a_hi, b_hi)


def _shift_up(x):
    return jnp.concatenate([x[1:], jnp.zeros((1, x.shape[1]), x.dtype)], 0)


def _shift_down(x):
    return jnp.concatenate([jnp.zeros((1, x.shape[1]), x.dtype), x[:-1]], 0)


def _frontend_kernel(xr_ref, cw_ref, sw_ref, fb_ref,
                     w1_ref, b1_ref, w2_ref, b2_ref, out_ref):
    v = xr_ref[0]
    frames = jnp.concatenate(
        [v[0:NFRAMES], v[1:NFRAMES + 1], v[2:NFRAMES + 2], v[3:NFRAMES + 3]],
        axis=1)
    re = jnp.dot(frames, cw_ref[...], preferred_element_type=jnp.float32,
                 precision=HI)
    im = jnp.dot(frames, sw_ref[...], preferred_element_type=jnp.float32,
                 precision=HI)
    mag = jnp.sqrt(re * re + im * im)
    spec = mag * mag
    mel = jnp.dot(spec, fb_ref[...], preferred_element_type=jnp.float32)
    x = jnp.log(jnp.clip(mel, 1e-5, None))
    xcat = jnp.concatenate([_shift_down(x), x, _shift_up(x)], axis=1)
    h = jnp.dot(xcat, w1_ref[...], preferred_element_type=jnp.float32) + b1_ref[...]
    h = 0.5 * h * (1.0 + jax.lax.erf(h * np.float32(0.7071067811865476)))
    hcat = jnp.concatenate([_shift_down(h), h, _shift_up(h)], axis=1)
    h2 = jnp.dot(hcat, w2_ref[...], preferred_element_type=jnp.float32) + b2_ref[...]
    out_ref[0] = 0.5 * h2 * (1.0 + jax.lax.erf(h2 * np.float32(0.7071067811865476)))


def _features(waveform, conv1_w, conv1_b, conv2_w, conv2_b):
    B = waveform.shape[0]
    pad = N_FFT // 2
    x = jnp.pad(waveform, ((0, 0), (pad, pad)), mode="reflect")
    xr = x.reshape(B, NHOPS, HOP)
    cw, sw, fb = _dft_consts()
    w1 = jnp.transpose(conv1_w, (2, 1, 0)).reshape(3 * N_MELS, 256)
    w2 = jnp.transpose(conv2_w, (2, 1, 0)).reshape(3 * 256, D_MODEL)
    b1 = conv1_b.reshape(1, -1)
    b2 = conv2_b.reshape(1, -1)
    feats = pl.pallas_call(
        _frontend_kernel,
        grid=(B,),
        in_specs=[
            pl.BlockSpec((1, NHOPS, HOP), lambda b: (b, 0, 0)),
            pl.BlockSpec((N_FFT, KFREQ), lambda b: (0, 0)),
            pl.BlockSpec((N_FFT, KFREQ), lambda b: (0, 0)),
            pl.BlockSpec((KFREQ, N_MELS), lambda b: (0, 0)),
            pl.BlockSpec((3 * N_MELS, 256), lambda b: (0, 0)),
            pl.BlockSpec((1, 256), lambda b: (0, 0)),
            pl.BlockSpec((3 * 256, D_MODEL), lambda b: (0, 0)),
            pl.BlockSpec((1, D_MODEL), lambda b: (0, 0)),
        ],
        out_specs=pl.BlockSpec((1, NFRAMES, D_MODEL), lambda b: (b, 0, 0)),
        out_shape=jax.ShapeDtypeStruct((B, NFRAMES, D_MODEL), jnp.float32),
        compiler_params=pltpu.CompilerParams(
            dimension_semantics=("arbitrary",)),
    )(xr, jnp.asarray(cw), jnp.asarray(sw), jnp.asarray(fb),
      w1, b1, w2, b2)
    return feats


def _rvq_kernel(flat_ref, cb_ref, b_ref, tok_ref, qsum_ref,
                res_ref, qcur_ref, best_ref, bidx_ref, a_ref):
    cb = pl.program_id(0)
    vc = pl.program_id(1)

    @pl.when(jnp.logical_and(cb == 0, vc == 0))
    def _init_residual():
        res_ref[...] = flat_ref[...]

    @pl.when(vc == 0)
    def _start_codebook():
        r = res_ref[...]
        a_ref[...] = jnp.sum(r * r, axis=1, keepdims=True)
        best_ref[...] = jnp.full((ROWS_PAD, 1), jnp.inf, jnp.float32)
        bidx_ref[...] = jnp.zeros((ROWS_PAD, 1), jnp.int32)

    r = res_ref[...]
    w = cb_ref[0]
    fw = jax.lax.dot_general(r, w, (((1,), (1,)), ((), ())),
                             preferred_element_type=jnp.float32)
    dist = (a_ref[...] - 2.0 * fw) + b_ref[0]

    m_c = jnp.min(dist, axis=1, keepdims=True)
    lane = jax.lax.broadcasted_iota(jnp.int32, (ROWS_PAD, VCHUNK), 1)
    idx_local = jnp.min(jnp.where(dist == m_c, lane, VCHUNK),
                        axis=1, keepdims=True)
    better = m_c < best_ref[...]
    onehot = (lane == idx_local).astype(jnp.bfloat16)
    w_hi = w.astype(jnp.bfloat16)
    r1 = w - w_hi.astype(jnp.float32)
    w_mid = r1.astype(jnp.bfloat16)
    w_lo = (r1 - w_mid.astype(jnp.float32)).astype(jnp.bfloat16)

    def g(ww):
        return jnp.dot(onehot, ww, preferred_element_type=jnp.float32)

    qnew = (g(w_hi) + g(w_mid)) + g(w_lo)
    qcur_ref[...] = jnp.where(better, qnew, qcur_ref[...])
    best_ref[...] = jnp.where(better, m_c, best_ref[...])
    bidx_ref[...] = jnp.where(better, idx_local + vc * VCHUNK, bidx_ref[...])

    @pl.when(vc == NVC - 1)
    def _finish_codebook():
        tok_ref[0] = bidx_ref[...]
        q = qcur_ref[...]
        res_ref[...] = res_ref[...] - q

        @pl.when(cb == 0)
        def _():
            qsum_ref[...] = q

        @pl.when(cb > 0)
        def _():
            qsum_ref[...] = qsum_ref[...] + q


def _rvq(flat, codebooks, bsq):
    tok, qsum = pl.pallas_call(
        _rvq_kernel,
        grid=(N_CB, NVC),
        in_specs=[
            pl.BlockSpec((ROWS_PAD, D_MODEL), lambda cb, vc: (0, 0)),
            pl.BlockSpec((1, VCHUNK, D_MODEL), lambda cb, vc: (cb, vc, 0)),
            pl.BlockSpec((1, 1, VCHUNK), lambda cb, vc: (cb * NVC + vc, 0, 0)),
        ],
        out_specs=[
            pl.BlockSpec((1, ROWS_PAD, 1), lambda cb, vc: (cb, 0, 0)),
            pl.BlockSpec((ROWS_PAD, D_MODEL), lambda cb, vc: (0, 0)),
        ],
        out_shape=[
            jax.ShapeDtypeStruct((N_CB, ROWS_PAD, 1), jnp.int32),
            jax.ShapeDtypeStruct((ROWS_PAD, D_MODEL), jnp.float32),
        ],
        scratch_shapes=[
            pltpu.VMEM((ROWS_PAD, D_MODEL), jnp.float32),
            pltpu.VMEM((ROWS_PAD, D_MODEL), jnp.float32),
            pltpu.VMEM((ROWS_PAD, 1), jnp.float32),
            pltpu.VMEM((ROWS_PAD, 1), jnp.int32),
            pltpu.VMEM((ROWS_PAD, 1), jnp.float32),
        ],
        compiler_params=pltpu.CompilerParams(
            dimension_semantics=("arbitrary", "arbitrary")),
    )(flat, codebooks, bsq)
    return tok, qsum


def kernel(waveform, conv1_w, conv1_b, conv2_w, conv2_b, codebooks):
    features = _features(waveform, conv1_w, conv1_b, conv2_w, conv2_b)
    B, L, D = features.shape
    flat = features.reshape(-1, D)
    nrows = B * L
    flat_p = jnp.pad(flat, ((0, ROWS_PAD - nrows), (0, 0)))
    bsq = jnp.sum(codebooks ** 2, axis=-1).reshape(N_CB * NVC, 1, VCHUNK)
    tok, qsum = _rvq(flat_p, codebooks, bsq)
    tokens = tok[:, :nrows, 0].reshape(N_CB, B, L).transpose(1, 0, 2)
    qsum = qsum[:nrows].reshape(B, L, D)
    return tokens, qsum


# speedup vs baseline: 1.1391x; 1.1391x over previous
import functools

import jax
import jax.numpy as jnp
import numpy as np
from jax.experimental import pallas as pl
from jax.experimental.pallas import tpu as pltpu

SR = 44100
N_FFT = 2048
HOP = 512
N_MELS = 128
D_MODEL = 512
VOCAB = 4096
N_CB = 4

ROWS_PAD = 1032
VCHUNK = 1024
NVC = VOCAB // VCHUNK


def _mel_filterbank():
    n_freqs = N_FFT // 2 + 1
    all_freqs = np.linspace(0.0, SR / 2.0, n_freqs)

    def hz2mel(f):
        return 2595.0 * np.log10(1.0 + f / 700.0)

    def mel2hz(m):
        return 700.0 * (10.0 ** (m / 2595.0) - 1.0)

    m_pts = np.linspace(hz2mel(0.0), hz2mel(SR / 2.0), N_MELS + 2)
    f_pts = mel2hz(m_pts)
    f_diff = f_pts[1:] - f_pts[:-1]
    slopes = f_pts[None, :] - all_freqs[:, None]
    down = -slopes[:, :-2] / f_diff[:-1]
    up = slopes[:, 2:] / f_diff[1:]
    fb = np.maximum(0.0, np.minimum(down, up))
    return fb.astype(np.float32)


NFRAMES = 257
NHOPS = 260
KFREQ = 1152

HI = jax.lax.Precision.HIGHEST


@functools.lru_cache(maxsize=1)
def _dft_consts():
    n = np.arange(N_FFT, dtype=np.float64)
    k = np.arange(KFREQ, dtype=np.float64)
    win = 0.5 - 0.5 * np.cos(2.0 * np.pi * n / N_FFT)
    ang = (2.0 * np.pi / N_FFT) * np.outer(n, k)
    cw = (win[:, None] * np.cos(ang)).astype(np.float32)
    sw = (win[:, None] * np.sin(ang)).astype(np.float32)
    cw[:, N_FFT // 2 + 1:] = 0.0
    sw[:, N_FFT // 2 + 1:] = 0.0
    fb = np.zeros((KFREQ, N_MELS), np.float32)
    fb[:N_FFT // 2 + 1] = _mel_filterbank()
    return cw, sw, fb


def _dot_split(a, b):
    a_hi = a.astype(jnp.bfloat16)
    a_lo = (a - a_hi.astype(jnp.float32)).astype(jnp.bfloat16)
    b_hi = b.astype(jnp.bfloat16)
    b_lo = (b - b_hi.astype(jnp.float32)).astype(jnp.bfloat16)

    def d(x, y):
        return jnp.dot(x, y, preferred_element_type=jnp.float32)

    return ((d(a_lo, b_lo) + d(a_lo, b_hi)) + d(a_hi, b_lo)) + d(a_hi, b_hi)


def _shift_up(x):
    return jnp.concatenate([x[1:], jnp.zeros((1, x.shape[1]), x.dtype)], 0)


def _shift_down(x):
    return jnp.concatenate([jnp.zeros((1, x.shape[1]), x.dtype), x[:-1]], 0)


def _frontend_kernel(xr_ref, cw_ref, sw_ref, fb_ref,
                     w1_ref, b1_ref, w2_ref, b2_ref, out_ref):
    v = xr_ref[0]
    frames = jnp.concatenate(
        [v[0:NFRAMES], v[1:NFRAMES + 1], v[2:NFRAMES + 2], v[3:NFRAMES + 3]],
        axis=1)
    re = jnp.dot(frames, cw_ref[...], preferred_element_type=jnp.float32,
                 precision=HI)
    im = jnp.dot(frames, sw_ref[...], preferred_element_type=jnp.float32,
                 precision=HI)
    mag = jnp.sqrt(re * re + im * im)
    spec = mag * mag
    mel = jnp.dot(spec, fb_ref[...], preferred_element_type=jnp.float32)
    x = jnp.log(jnp.clip(mel, 1e-5, None))
    xcat = jnp.concatenate([_shift_down(x), x, _shift_up(x)], axis=1)
    h = jnp.dot(xcat, w1_ref[...], preferred_element_type=jnp.float32) + b1_ref[...]
    h = 0.5 * h * (1.0 + jax.lax.erf(h * np.float32(0.7071067811865476)))
    hcat = jnp.concatenate([_shift_down(h), h, _shift_up(h)], axis=1)
    h2 = jnp.dot(hcat, w2_ref[...], preferred_element_type=jnp.float32) + b2_ref[...]
    out_ref[0] = 0.5 * h2 * (1.0 + jax.lax.erf(h2 * np.float32(0.7071067811865476)))


def _features(waveform, conv1_w, conv1_b, conv2_w, conv2_b):
    B = waveform.shape[0]
    pad = N_FFT // 2
    x = jnp.pad(waveform, ((0, 0), (pad, pad)), mode="reflect")
    xr = x.reshape(B, NHOPS, HOP)
    cw, sw, fb = _dft_consts()
    w1 = jnp.transpose(conv1_w, (2, 1, 0)).reshape(3 * N_MELS, 256)
    w2 = jnp.transpose(conv2_w, (2, 1, 0)).reshape(3 * 256, D_MODEL)
    b1 = conv1_b.reshape(1, -1)
    b2 = conv2_b.reshape(1, -1)
    feats = pl.pallas_call(
        _frontend_kernel,
        grid=(B,),
        in_specs=[
            pl.BlockSpec((1, NHOPS, HOP), lambda b: (b, 0, 0)),
            pl.BlockSpec((N_FFT, KFREQ), lambda b: (0, 0)),
            pl.BlockSpec((N_FFT, KFREQ), lambda b: (0, 0)),
            pl.BlockSpec((KFREQ, N_MELS), lambda b: (0, 0)),
            pl.BlockSpec((3 * N_MELS, 256), lambda b: (0, 0)),
            pl.BlockSpec((1, 256), lambda b: (0, 0)),
            pl.BlockSpec((3 * 256, D_MODEL), lambda b: (0, 0)),
            pl.BlockSpec((1, D_MODEL), lambda b: (0, 0)),
        ],
        out_specs=pl.BlockSpec((1, NFRAMES, D_MODEL), lambda b: (b, 0, 0)),
        out_shape=jax.ShapeDtypeStruct((B, NFRAMES, D_MODEL), jnp.float32),
        compiler_params=pltpu.CompilerParams(
            dimension_semantics=("arbitrary",)),
    )(xr, jnp.asarray(cw), jnp.asarray(sw), jnp.asarray(fb),
      w1, b1, w2, b2)
    return feats


def _rvq_kernel(flat_ref, cb_ref, b_ref, tok_ref, qsum_ref,
                res_ref, qcur_ref, best_ref, bidx_ref, a_ref):
    cb = pl.program_id(0)
    vc = pl.program_id(1)

    @pl.when(jnp.logical_and(cb == 0, vc == 0))
    def _init_residual():
        res_ref[...] = flat_ref[...]

    @pl.when(vc == 0)
    def _start_codebook():
        r = res_ref[...]
        a_ref[...] = jnp.sum(r * r, axis=1, keepdims=True)
        best_ref[...] = jnp.full((ROWS_PAD, 1), jnp.inf, jnp.float32)
        bidx_ref[...] = jnp.zeros((ROWS_PAD, 1), jnp.int32)

    r = res_ref[...]
    w = cb_ref[0]
    fw = jax.lax.dot_general(r, w, (((1,), (1,)), ((), ())),
                             preferred_element_type=jnp.float32)
    dist = (a_ref[...] - 2.0 * fw) + b_ref[0]

    m_c = jnp.min(dist, axis=1, keepdims=True)
    lane = jax.lax.broadcasted_iota(jnp.int32, (ROWS_PAD, VCHUNK), 1)
    idx_local = jnp.min(jnp.where(dist == m_c, lane, VCHUNK),
                        axis=1, keepdims=True)
    better = m_c < best_ref[...]
    onehot = (lane == idx_local).astype(jnp.bfloat16)
    w_hi = w.astype(jnp.bfloat16)
    w_mid = (w - w_hi.astype(jnp.float32)).astype(jnp.bfloat16)

    def g(ww):
        return jnp.dot(onehot, ww, preferred_element_type=jnp.float32)

    qnew = g(w_hi) + g(w_mid)
    qcur_ref[...] = jnp.where(better, qnew, qcur_ref[...])
    best_ref[...] = jnp.where(better, m_c, best_ref[...])
    bidx_ref[...] = jnp.where(better, idx_local + vc * VCHUNK, bidx_ref[...])

    @pl.when(vc == NVC - 1)
    def _finish_codebook():
        tok_ref[0] = bidx_ref[...]
        q = qcur_ref[...]
        res_ref[...] = res_ref[...] - q

        @pl.when(cb == 0)
        def _():
            qsum_ref[...] = q

        @pl.when(cb > 0)
        def _():
            qsum_ref[...] = qsum_ref[...] + q


def _rvq(flat, codebooks, bsq):
    tok, qsum = pl.pallas_call(
        _rvq_kernel,
        grid=(N_CB, NVC),
        in_specs=[
            pl.BlockSpec((ROWS_PAD, D_MODEL), lambda cb, vc: (0, 0)),
            pl.BlockSpec((1, VCHUNK, D_MODEL), lambda cb, vc: (cb, vc, 0)),
            pl.BlockSpec((1, 1, VCHUNK), lambda cb, vc: (cb * NVC + vc, 0, 0)),
        ],
        out_specs=[
            pl.BlockSpec((1, ROWS_PAD, 1), lambda cb, vc: (cb, 0, 0)),
            pl.BlockSpec((ROWS_PAD, D_MODEL), lambda cb, vc: (0, 0)),
        ],
        out_shape=[
            jax.ShapeDtypeStruct((N_CB, ROWS_PAD, 1), jnp.int32),
            jax.ShapeDtypeStruct((ROWS_PAD, D_MODEL), jnp.float32),
        ],
        scratch_shapes=[
            pltpu.VMEM((ROWS_PAD, D_MODEL), jnp.float32),
            pltpu.VMEM((ROWS_PAD, D_MODEL), jnp.float32),
            pltpu.VMEM((ROWS_PAD, 1), jnp.float32),
            pltpu.VMEM((ROWS_PAD, 1), jnp.int32),
            pltpu.VMEM((ROWS_PAD, 1), jnp.float32),
        ],
        compiler_params=pltpu.CompilerParams(
            dimension_semantics=("arbitrary", "arbitrary")),
    )(flat, codebooks, bsq)
    return tok, qsum


def kernel(waveform, conv1_w, conv1_b, conv2_w, conv2_b, codebooks):
    features = _features(waveform, conv1_w, conv1_b, conv2_w, conv2_b)
    B, L, D = features.shape
    flat = features.reshape(-1, D)
    nrows = B * L
    flat_p = jnp.pad(flat, ((0, ROWS_PAD - nrows), (0, 0)))
    bsq = jnp.sum(codebooks ** 2, axis=-1).reshape(N_CB * NVC, 1, VCHUNK)
    tok, qsum = _rvq(flat_p, codebooks, bsq)
    tokens = tok[:, :nrows, 0].reshape(N_CB, B, L).transpose(1, 0, 2)
    qsum = qsum[:nrows].reshape(B, L, D)
    return tokens, qsum
